# baseline (speedup 1.0000x reference)
import jax
import jax.numpy as jnp
from jax import lax
from jax.experimental import pallas as pl
from jax.experimental.pallas import tpu as pltpu
from jax.experimental.pallas import tpu_sc as plsc

B, S, F = 4, 2048, 4096
F_TC = 3072
F_SC = F - F_TC

FW = 128
CH = 64
NG = FW // 16
NCHUNK = S // CH
NBUF = 4
UNROLL = 8

SB = 256
FB = 1024


def _sc_body(x_hbm, o_hbm,
             in0, in1, in2, in3, out0, out1, out2, out3,
             isem0, isem1, isem2, isem3, osem0, osem1, osem2, osem3):
    cid = lax.axis_index("c")
    sid = lax.axis_index("s")
    wid = sid * 2 + cid
    b = wid // 8
    f_base = F_TC + (wid % 8) * FW

    ins = (in0, in1, in2, in3)
    outs = (out0, out1, out2, out3)
    isems = (isem0, isem1, isem2, isem3)
    osems = (osem0, osem1, osem2, osem3)

    def start_in(slot, t):
        pltpu.async_copy(
            x_hbm.at[b, pl.ds(t * CH, CH), pl.ds(f_base, FW)],
            ins[slot], isems[slot],
        )

    def start_out(slot, t):
        pltpu.async_copy(
            outs[slot],
            o_hbm.at[b, pl.ds(t * CH, CH), pl.ds(f_base, FW)],
            osems[slot],
        )

    def wait_in(slot):
        pltpu.make_async_copy(x_hbm.at[b, pl.ds(0, CH), pl.ds(0, FW)],
                              ins[slot], isems[slot]).wait()

    def wait_out(slot):
        pltpu.make_async_copy(outs[slot],
                              o_hbm.at[b, pl.ds(0, CH), pl.ds(0, FW)],
                              osems[slot]).wait()

    for slot in range(NBUF):
        start_in(slot, slot)

    def ring_body(i4, carries):
        for par in range(NBUF):
            t = i4 * NBUF + par
            wait_in(par)

            @pl.when(t >= NBUF)
            def _():
                wait_out(par)

            def s_body(sj, carr):
                for u in range(UNROLL):
                    si = sj * UNROLL + u
                    nxt = []
                    for g in range(NG):
                        v = ins[par][si, pl.ds(g * 16, 16)]
                        nc = carr[g] + v
                        outs[par][si, pl.ds(g * 16, 16)] = nc
                        nxt.append(nc)
                    carr = tuple(nxt)
                return carr

            carries = lax.fori_loop(0, CH // UNROLL, s_body, carries)
            start_out(par, t)

            @pl.when(t + NBUF < NCHUNK)
            def _():
                start_in(par, t + NBUF)
        return carries

    zero = jnp.zeros((16,), jnp.float32)
    lax.fori_loop(0, NCHUNK // NBUF, ring_body, tuple(zero for _ in range(NG)))
    for slot in range(NBUF):
        wait_out(slot)


def _sc_slice(x):
    mesh = plsc.VectorSubcoreMesh(core_axis_name="c", subcore_axis_name="s")
    kfn = pl.kernel(
        _sc_body,
        mesh=mesh,
        out_type=jax.ShapeDtypeStruct((B, S, F), jnp.float32),
        scratch_types=(
            [pltpu.VMEM((CH, FW), jnp.float32)] * 8
            + [pltpu.SemaphoreType.DMA] * 8
        ),
    )
    return kfn(x)


def _tc_body(x_ref, part_ref, o_ref, carry_ref):
    s = pl.program_id(2)

    @pl.when(s == 0)
    def _():
        carry_ref[...] = jnp.zeros_like(carry_ref)

    xb = x_ref[0]
    row = jax.lax.broadcasted_iota(jnp.int32, (SB, SB), 0)
    col = jax.lax.broadcasted_iota(jnp.int32, (SB, SB), 1)
    tri = (row >= col).astype(jnp.float32)
    part = jnp.dot(tri, xb, preferred_element_type=jnp.float32)
    y = part + carry_ref[...]
    o_ref[0] = y
    carry_ref[...] = y[SB - 1 : SB, :]


def kernel(x):
    partial = _sc_slice(x)
    grid = (B, F_TC // FB, S // SB)
    return pl.pallas_call(
        _tc_body,
        grid=grid,
        in_specs=[
            pl.BlockSpec((1, SB, FB), lambda b, f, s: (b, s, f)),
            pl.BlockSpec((1, 8, 128), lambda b, f, s: (0, 0, 0)),
        ],
        out_specs=pl.BlockSpec((1, SB, FB), lambda b, f, s: (b, s, f)),
        out_shape=jax.ShapeDtypeStruct((B, S, F), jnp.float32),
        scratch_shapes=[pltpu.VMEM((1, FB), jnp.float32)],
        input_output_aliases={1: 0},
        compiler_params=pltpu.CompilerParams(
            dimension_semantics=("parallel", "parallel", "arbitrary"),
        ),
    )(x, partial)


# speedup vs baseline: 1.2795x; 1.2795x over previous
import jax
import jax.numpy as jnp
from jax import lax
from jax.experimental import pallas as pl
from jax.experimental.pallas import tpu as pltpu
from jax.experimental.pallas import tpu_sc as plsc

B, S, F = 4, 2048, 4096
F_TC = 2048
F_SC = F - F_TC

FW = 128
CH = 64
NG = FW // 16
NCHUNK = S // CH
NSTRIP = F_SC // 8 // FW
T = NSTRIP * NCHUNK
NBUF = 4
UNROLL = 8

SB = 256
FB = 2048


def _sc_body(x_hbm, o_hbm,
             in0, in1, in2, in3, out0, out1, out2, out3,
             isem0, isem1, isem2, isem3, osem0, osem1, osem2, osem3):
    cid = lax.axis_index("c")
    sid = lax.axis_index("s")
    wid = sid * 2 + cid
    b = wid // 8
    f_base = F_TC + (wid % 8) * (F_SC // 8)

    ins = (in0, in1, in2, in3)
    outs = (out0, out1, out2, out3)
    isems = (isem0, isem1, isem2, isem3)
    osems = (osem0, osem1, osem2, osem3)

    def src_at(t):
        k = t // NCHUNK
        ci = lax.rem(t, NCHUNK)
        return ci * CH, f_base + k * FW

    def start_in(slot, t):
        s0, f0 = src_at(t)
        pltpu.async_copy(
            x_hbm.at[b, pl.ds(s0, CH), pl.ds(f0, FW)],
            ins[slot], isems[slot],
        )

    def start_out(slot, t):
        s0, f0 = src_at(t)
        pltpu.async_copy(
            outs[slot],
            o_hbm.at[b, pl.ds(s0, CH), pl.ds(f0, FW)],
            osems[slot],
        )

    def wait_in(slot):
        pltpu.make_async_copy(x_hbm.at[b, pl.ds(0, CH), pl.ds(0, FW)],
                              ins[slot], isems[slot]).wait()

    def wait_out(slot):
        pltpu.make_async_copy(outs[slot],
                              o_hbm.at[b, pl.ds(0, CH), pl.ds(0, FW)],
                              osems[slot]).wait()

    for slot in range(NBUF):
        start_in(slot, slot)

    def ring_body(i4, carries):
        for par in range(NBUF):
            t = i4 * NBUF + par
            ci = lax.rem(t, NCHUNK)
            wait_in(par)

            @pl.when(t >= NBUF)
            def _():
                wait_out(par)

            zero = jnp.zeros((16,), jnp.float32)
            carries = tuple(jnp.where(ci == 0, zero, c) for c in carries)

            def s_body(sj, carr):
                for u in range(UNROLL):
                    si = sj * UNROLL + u
                    nxt = []
                    for g in range(NG):
                        v = ins[par][si, pl.ds(g * 16, 16)]
                        nc = carr[g] + v
                        outs[par][si, pl.ds(g * 16, 16)] = nc
                        nxt.append(nc)
                    carr = tuple(nxt)
                return carr

            carries = lax.fori_loop(0, CH // UNROLL, s_body, carries)
            start_out(par, t)

            @pl.when(t + NBUF < T)
            def _():
                start_in(par, t + NBUF)
        return carries

    zero = jnp.zeros((16,), jnp.float32)
    lax.fori_loop(0, T // NBUF, ring_body, tuple(zero for _ in range(NG)))
    for slot in range(NBUF):
        wait_out(slot)


def _sc_slice(x):
    mesh = plsc.VectorSubcoreMesh(core_axis_name="c", subcore_axis_name="s")
    kfn = pl.kernel(
        _sc_body,
        mesh=mesh,
        out_type=jax.ShapeDtypeStruct((B, S, F), jnp.float32),
        scratch_types=(
            [pltpu.VMEM((CH, FW), jnp.float32)] * 8
            + [pltpu.SemaphoreType.DMA] * 8
        ),
    )
    return kfn(x)


def _tc_body(x_ref, part_ref, o_ref, carry_ref):
    s = pl.program_id(2)

    @pl.when(s == 0)
    def _():
        carry_ref[...] = jnp.zeros_like(carry_ref)

    xb = x_ref[0]
    row = jax.lax.broadcasted_iota(jnp.int32, (SB, SB), 0)
    col = jax.lax.broadcasted_iota(jnp.int32, (SB, SB), 1)
    tri = (row >= col).astype(jnp.float32)
    part = jnp.dot(tri, xb, preferred_element_type=jnp.float32)
    y = part + carry_ref[...]
    o_ref[0] = y
    carry_ref[...] = y[SB - 1 : SB, :]


def kernel(x):
    partial = _sc_slice(x)
    grid = (B, F_TC // FB, S // SB)
    return pl.pallas_call(
        _tc_body,
        grid=grid,
        in_specs=[
            pl.BlockSpec((1, SB, FB), lambda b, f, s: (b, s, f)),
            pl.BlockSpec((1, 8, 128), lambda b, f, s: (0, 0, 0)),
        ],
        out_specs=pl.BlockSpec((1, SB, FB), lambda b, f, s: (b, s, f)),
        out_shape=jax.ShapeDtypeStruct((B, S, F), jnp.float32),
        scratch_shapes=[pltpu.VMEM((1, FB), jnp.float32)],
        input_output_aliases={1: 0},
        compiler_params=pltpu.CompilerParams(
            dimension_semantics=("parallel", "parallel", "arbitrary"),
        ),
    )(x, partial)


# speedup vs baseline: 1.3167x; 1.0290x over previous
import jax
import jax.numpy as jnp
from jax import lax
from jax.experimental import pallas as pl
from jax.experimental.pallas import tpu as pltpu
from jax.experimental.pallas import tpu_sc as plsc

B, S, F = 4, 2048, 4096
FW = 128
CH = 64
NG = FW // 16
F_PER_W = F // 8
NSTRIP = F_PER_W // FW
NCHUNK = S // CH
T = NSTRIP * NCHUNK
NBUF = 4
UNROLL = 8


def _sc_body(x_hbm, o_hbm,
             in0, in1, in2, in3, out0, out1, out2, out3,
             isem0, isem1, isem2, isem3, osem0, osem1, osem2, osem3):
    cid = lax.axis_index("c")
    sid = lax.axis_index("s")
    wid = sid * 2 + cid
    b = wid // 8
    f_base = (wid % 8) * F_PER_W

    ins = (in0, in1, in2, in3)
    outs = (out0, out1, out2, out3)
    isems = (isem0, isem1, isem2, isem3)
    osems = (osem0, osem1, osem2, osem3)

    def src_at(t):
        k = t // NCHUNK
        ci = lax.rem(t, NCHUNK)
        return ci * CH, f_base + k * FW

    def start_in(slot, t):
        s0, f0 = src_at(t)
        pltpu.async_copy(
            x_hbm.at[b, pl.ds(s0, CH), pl.ds(f0, FW)], ins[slot], isems[slot]
        )

    def start_out(slot, t):
        s0, f0 = src_at(t)
        pltpu.async_copy(
            outs[slot], o_hbm.at[b, pl.ds(s0, CH), pl.ds(f0, FW)], osems[slot]
        )

    def wait_in(slot):
        pltpu.make_async_copy(x_hbm.at[b, pl.ds(0, CH), pl.ds(0, FW)],
                              ins[slot], isems[slot]).wait()

    def wait_out(slot):
        pltpu.make_async_copy(outs[slot],
                              o_hbm.at[b, pl.ds(0, CH), pl.ds(0, FW)],
                              osems[slot]).wait()

    for slot in range(NBUF):
        start_in(slot, slot)

    def ring_body(i4, carries):
        for par in range(NBUF):
            t = i4 * NBUF + par
            ci = lax.rem(t, NCHUNK)
            wait_in(par)

            @pl.when(t >= NBUF)
            def _():
                wait_out(par)

            zero = jnp.zeros((16,), jnp.float32)
            carries = tuple(jnp.where(ci == 0, zero, c) for c in carries)

            def s_body(sj, carr):
                for u in range(UNROLL):
                    si = sj * UNROLL + u
                    nxt = []
                    for g in range(NG):
                        v = ins[par][si, pl.ds(g * 16, 16)]
                        nc = carr[g] + v
                        outs[par][si, pl.ds(g * 16, 16)] = nc
                        nxt.append(nc)
                    carr = tuple(nxt)
                return carr

            carries = lax.fori_loop(0, CH // UNROLL, s_body, carries)
            start_out(par, t)

            @pl.when(t + NBUF < T)
            def _():
                start_in(par, t + NBUF)
        return carries

    zero = jnp.zeros((16,), jnp.float32)
    lax.fori_loop(0, T // NBUF, ring_body, tuple(zero for _ in range(NG)))
    for slot in range(NBUF):
        wait_out(slot)


def kernel(x):
    mesh = plsc.VectorSubcoreMesh(core_axis_name="c", subcore_axis_name="s")
    kfn = pl.kernel(
        _sc_body,
        mesh=mesh,
        out_type=jax.ShapeDtypeStruct((B, S, F), jnp.float32),
        scratch_types=(
            [pltpu.VMEM((CH, FW), jnp.float32)] * 8
            + [pltpu.SemaphoreType.DMA] * 8
        ),
    )
    return kfn(x)
